# baseline (speedup 1.0000x reference)
import math

import jax
import jax.numpy as jnp
from jax.experimental import pallas as pl

B = 1
N = 4096
D = 1024
H = 16
DH = 64
DFF = 4096
KK = 512
EPS = 1e-6
F32 = jnp.float32
HI = jax.lax.Precision.HIGHEST


def _ln(x, g, b):
    mu = jnp.mean(x, axis=-1, keepdims=True)
    var = jnp.mean((x - mu) ** 2, axis=-1, keepdims=True)
    return g * ((x - mu) / (jnp.sqrt(var) + EPS)) + b


def _select_kernel(c_col_ref, c_row_ref, idx_ref):
    c_row = c_row_ref[...]
    jidx = jax.lax.broadcasted_iota(jnp.int32, (1, N), 1)
    CH = 512

    chunks = []
    for ic in range(N // CH):
        ci = c_col_ref[ic * CH:(ic + 1) * CH, :]
        ii = jax.lax.broadcasted_iota(jnp.int32, (CH, 1), 0) + ic * CH
        gt = (c_row > ci).astype(jnp.int32)
        eq = ((c_row == ci) & (jidx < ii)).astype(jnp.int32)
        chunks.append(jnp.sum(gt + eq, axis=1, keepdims=True))
    ranks = jnp.concatenate(chunks, axis=0)
    rvals = jax.lax.broadcasted_iota(jnp.int32, (1, KK), 1)
    oh = (ranks == rvals).astype(F32)
    ivals = jax.lax.broadcasted_iota(jnp.int32, (1, N), 1).astype(F32)
    idxf = jnp.dot(ivals, oh, preferred_element_type=F32, precision=HI)
    idx_ref[...] = idxf.astype(jnp.int32)


def _kv_kernel(x_ref, g_ref, b_ref, wk_ref, bk_ref, wv_ref, bv_ref, k_ref, v_ref):
    xn = _ln(x_ref[...], g_ref[...], b_ref[...])
    k_ref[...] = jnp.dot(xn, wk_ref[...], preferred_element_type=F32, precision=HI) + bk_ref[...]
    v_ref[...] = jnp.dot(xn, wv_ref[...], preferred_element_type=F32, precision=HI) + bv_ref[...]


def _q_kernel(idx_ref, x_ref, g_ref, b_ref, wq_ref, bq_ref, q_ref, xsel_ref):
    idxc = idx_ref[...]
    CH = 512
    xsel = jnp.zeros((KK, D), F32)
    for ic in range(N // CH):
        iota_r = jax.lax.broadcasted_iota(jnp.int32, (KK, CH), 1) + ic * CH
        oh = (iota_r == idxc).astype(F32)
        xsel = xsel + jnp.dot(oh, x_ref[ic * CH:(ic + 1) * CH, :],
                              preferred_element_type=F32, precision=HI)
    xn = _ln(xsel, g_ref[...], b_ref[...])
    q_ref[...] = jnp.dot(xn, wq_ref[...], preferred_element_type=F32, precision=HI) + bq_ref[...]
    xsel_ref[...] = xsel


def _attn_kernel(q_ref, k_ref, v_ref, p_ref, ctx_ref):
    q = q_ref[0]
    k = k_ref[0]
    s = jax.lax.dot_general(q, k, (((1,), (1,)), ((), ())),
                            preferred_element_type=F32, precision=HI)
    s = s * (1.0 / math.sqrt(DH))
    m = jnp.max(s, axis=-1, keepdims=True)
    e = jnp.exp(s - m)
    p = e / jnp.sum(e, axis=-1, keepdims=True)
    p_ref[0] = p
    ctx_ref[0] = jnp.dot(p, v_ref[0], preferred_element_type=F32, precision=HI)


def _outproj_kernel(ctx_ref, wo_ref, bo_ref, xsel_ref, xbig_ref):
    xbig_ref[...] = (jnp.dot(ctx_ref[...], wo_ref[...],
                             preferred_element_type=F32, precision=HI)
                     + bo_ref[...] + xsel_ref[...])


def _scatter_kernel(xbig_ref, idxr_ref, x_ref, y_ref):
    i = pl.program_id(0)
    SB = 512
    iota_c = jax.lax.broadcasted_iota(jnp.int32, (SB, KK), 0) + i * SB
    oh_t = (iota_c == idxr_ref[...]).astype(F32)
    scat = jnp.dot(oh_t, xbig_ref[...], preferred_element_type=F32, precision=HI)
    msk = jnp.sum(oh_t, axis=1, keepdims=True)
    y_ref[...] = jnp.where(msk > 0, scat, x_ref[...])


def _ffn_kernel(y_ref, g_ref, b_ref, w1_ref, b1_ref, w2_ref, b2_ref, o_ref):
    yb = y_ref[...]
    xn = _ln(yb, g_ref[...], b_ref[...])
    h = jnp.maximum(jnp.dot(xn, w1_ref[...], preferred_element_type=F32, precision=HI)
                    + b1_ref[...], 0.0)
    o_ref[...] = yb + jnp.dot(h, w2_ref[...], preferred_element_type=F32, precision=HI) + b2_ref[...]


def _full(shape):
    nd = len(shape)
    return pl.BlockSpec(shape, lambda *args: (0,) * nd)


def kernel(x, c, Wq, bq, Wk, bk, Wv, bv, Wo, bo, W1, b1, W2, b2, g1, be1, g2, be2):
    x2 = x.reshape(N, D)
    c_col = c.reshape(N, 1)
    c_row = c.reshape(1, N)
    g1r = g1.reshape(1, D)
    be1r = be1.reshape(1, D)
    g2r = g2.reshape(1, D)
    be2r = be2.reshape(1, D)
    bqr = bq.reshape(1, D)
    bkr = bk.reshape(1, D)
    bvr = bv.reshape(1, D)
    bor = bo.reshape(1, D)
    b1r = b1.reshape(1, DFF)
    b2r = b2.reshape(1, D)

    idx = pl.pallas_call(
        _select_kernel,
        out_shape=jax.ShapeDtypeStruct((1, KK), jnp.int32),
        in_specs=[_full((N, 1)), _full((1, N))],
        out_specs=_full((1, KK)),
    )(c_col, c_row)

    NB = 16
    RB = N // NB
    k2, v2 = pl.pallas_call(
        _kv_kernel,
        grid=(NB,),
        out_shape=[jax.ShapeDtypeStruct((N, D), F32),
                   jax.ShapeDtypeStruct((N, D), F32)],
        in_specs=[
            pl.BlockSpec((RB, D), lambda i: (i, 0)),
            pl.BlockSpec((1, D), lambda i: (0, 0)),
            pl.BlockSpec((1, D), lambda i: (0, 0)),
            pl.BlockSpec((D, D), lambda i: (0, 0)),
            pl.BlockSpec((1, D), lambda i: (0, 0)),
            pl.BlockSpec((D, D), lambda i: (0, 0)),
            pl.BlockSpec((1, D), lambda i: (0, 0)),
        ],
        out_specs=[pl.BlockSpec((RB, D), lambda i: (i, 0)),
                   pl.BlockSpec((RB, D), lambda i: (i, 0))],
    )(x2, g1r, be1r, Wk, bkr, Wv, bvr)

    q2, xsel = pl.pallas_call(
        _q_kernel,
        out_shape=[jax.ShapeDtypeStruct((KK, D), F32),
                   jax.ShapeDtypeStruct((KK, D), F32)],
        in_specs=[_full((KK, 1)), _full((N, D)), _full((1, D)), _full((1, D)),
                  _full((D, D)), _full((1, D))],
        out_specs=[_full((KK, D)), _full((KK, D))],
    )(idx.reshape(KK, 1), x2, g1r, be1r, Wq, bqr)

    q3 = q2.reshape(KK, H, DH).transpose(1, 0, 2)
    k3 = k2.reshape(N, H, DH).transpose(1, 0, 2)
    v3 = v2.reshape(N, H, DH).transpose(1, 0, 2)

    attn, ctx3 = pl.pallas_call(
        _attn_kernel,
        grid=(H,),
        out_shape=[jax.ShapeDtypeStruct((H, KK, N), F32),
                   jax.ShapeDtypeStruct((H, KK, DH), F32)],
        in_specs=[
            pl.BlockSpec((1, KK, DH), lambda h: (h, 0, 0)),
            pl.BlockSpec((1, N, DH), lambda h: (h, 0, 0)),
            pl.BlockSpec((1, N, DH), lambda h: (h, 0, 0)),
        ],
        out_specs=[pl.BlockSpec((1, KK, N), lambda h: (h, 0, 0)),
                   pl.BlockSpec((1, KK, DH), lambda h: (h, 0, 0))],
    )(q3, k3, v3)

    ctx2 = ctx3.transpose(1, 0, 2).reshape(KK, D)

    xbig = pl.pallas_call(
        _outproj_kernel,
        out_shape=jax.ShapeDtypeStruct((KK, D), F32),
        in_specs=[_full((KK, D)), _full((D, D)), _full((1, D)), _full((KK, D))],
        out_specs=_full((KK, D)),
    )(ctx2, Wo, bor, xsel)

    SB = 512
    y2 = pl.pallas_call(
        _scatter_kernel,
        grid=(N // SB,),
        out_shape=jax.ShapeDtypeStruct((N, D), F32),
        in_specs=[
            pl.BlockSpec((KK, D), lambda i: (0, 0)),
            pl.BlockSpec((1, KK), lambda i: (0, 0)),
            pl.BlockSpec((SB, D), lambda i: (i, 0)),
        ],
        out_specs=pl.BlockSpec((SB, D), lambda i: (i, 0)),
    )(xbig, idx, x2)

    out2 = pl.pallas_call(
        _ffn_kernel,
        grid=(NB,),
        out_shape=jax.ShapeDtypeStruct((N, D), F32),
        in_specs=[
            pl.BlockSpec((RB, D), lambda i: (i, 0)),
            pl.BlockSpec((1, D), lambda i: (0, 0)),
            pl.BlockSpec((1, D), lambda i: (0, 0)),
            pl.BlockSpec((D, DFF), lambda i: (0, 0)),
            pl.BlockSpec((1, DFF), lambda i: (0, 0)),
            pl.BlockSpec((DFF, D), lambda i: (0, 0)),
            pl.BlockSpec((1, D), lambda i: (0, 0)),
        ],
        out_specs=pl.BlockSpec((RB, D), lambda i: (i, 0)),
    )(y2, g2r, be2r, W1, b1r, W2, b2r)

    return (out2.reshape(B, N, D), attn.reshape(B, H, KK, N))
